# baseline (speedup 1.0000x reference)
import functools

import numpy as np

import jax
import jax.numpy as jnp
from jax import lax
from jax.experimental import pallas as pl
from jax.experimental.pallas import tpu as pltpu
from jax.experimental.pallas import tpu_sc as plsc

_VOCAB = 42
_VPAD = 48
_D = 128
_L = 200
_B = 1024
_NC, _NS = 2, 16
_NW = _NC * _NS
_ROWS = _B * _L
_RPW = _ROWS // _NW
_K = 128
_NCHUNK = _RPW // _K
_NB = 2
_E = 4


def _pos_encoding_np(max_seq, d_model):
    even_i = np.arange(0, d_model, 2, dtype=np.float32)
    denominator = np.power(np.float32(10000.0), even_i / np.float32(d_model))
    position = np.arange(max_seq, dtype=np.float32).reshape(max_seq, 1)
    even_pe = np.sin(position / denominator, dtype=np.float32)
    odd_pe = np.cos(position / denominator, dtype=np.float32)
    stacked = np.stack([even_pe, odd_pe], axis=2)
    return stacked.reshape(max_seq, d_model).astype(np.float32)


_PE = _pos_encoding_np(_L, _D)


def _combine_body(pe_ref, tab_ref, c_ref):
    pe = pe_ref[...]
    tab = tab_ref[...]
    c_ref[:, pl.ds(0, _VOCAB), :] = pe[:, None, :] + tab[None, :, :]


def _build_combined(pe, tab):
    c = pl.pallas_call(
        _combine_body,
        out_shape=jax.ShapeDtypeStruct((_L, _VPAD, _D), jnp.float32),
    )(pe, tab)
    return c.reshape(_L * _VPAD, _D)


@functools.partial(
    pl.kernel,
    out_type=jax.ShapeDtypeStruct((_ROWS, _D), jnp.float32),
    mesh=plsc.VectorSubcoreMesh(
        core_axis_name="c", subcore_axis_name="s",
        num_cores=_NC, num_subcores=_NS),
    scratch_types=[
        pltpu.VMEM((_RPW,), jnp.int32),
        pltpu.VMEM((_NB, 1, 128), jnp.int32),
        pltpu.VMEM((_NB, _K, _D), jnp.float32),
        pltpu.VMEM_SHARED((_L * _VPAD, _D), jnp.float32),
        pltpu.SemaphoreType.DMA,
        pltpu.SemaphoreType.DMA,
        pltpu.SemaphoreType.DMA,
        pltpu.SemaphoreType.DMA,
        pltpu.SemaphoreType.DMA,
    ],
)
def _sc_gather(tok_hbm, c_hbm, out_hbm, tok_v, idx_v, rows_v, c_sp,
               g0, g1, s0, s1, st):
    wid = lax.axis_index("s") * _NC + lax.axis_index("c")
    wbase = wid * _RPW
    g_sem = (g0, g1)
    s_sem = (s0, s1)

    sid = lax.axis_index("s")
    tok_cp = pltpu.make_async_copy(
        tok_hbm.at[pl.ds(wbase, _RPW)], tok_v, g_sem[0])
    tok_cp.start()

    rows_per_sub = (_L * _VPAD) // _NS
    stage_cp = pltpu.make_async_copy(
        c_hbm.at[pl.ds(sid * rows_per_sub, rows_per_sub)],
        c_sp.at[pl.ds(sid * rows_per_sub, rows_per_sub)], st)
    stage_cp.start()
    tok_cp.wait()

    def _gather_hbm_desc(i, b):
        return pltpu.make_async_copy(
            c_hbm.at[idx_v.at[b].at[0]], rows_v.at[b], g_sem[b])

    def _gather_desc(i, b):
        return pltpu.make_async_copy(
            c_sp.at[idx_v.at[b].at[0]], rows_v.at[b], g_sem[b])

    def _scatter_desc(i, b):
        return pltpu.make_async_copy(
            rows_v.at[b], out_hbm.at[pl.ds(wbase + i * _K, _K)], s_sem[b])

    def step(i, carry):
        def stage(b):
            @pl.when(jnp.logical_and(i >= _NB, i < _NCHUNK))
            def _():
                _scatter_desc(i - _NB, b).wait()

            @pl.when(i == _E)
            def _():
                stage_cp.wait()
                plsc.subcore_barrier()

            @pl.when(i < _NCHUNK)
            def _():
                for j in range(8):
                    o = i * _K + j * 16
                    pos = wbase + o + lax.iota(jnp.int32, 16)
                    l = lax.rem(pos, _L)
                    idx_v[b, 0, pl.ds(j * 16, 16)] = (
                        l * _VPAD + tok_v[pl.ds(o, 16)])

                @pl.when(i < _E)
                def _():
                    _gather_hbm_desc(i, b).start()

                @pl.when(i >= _E)
                def _():
                    _gather_desc(i, b).start()

            @pl.when(i >= 1)
            def _():
                bp = (b + _NB - 1) % _NB
                _gather_desc(i - 1, bp).wait()
                _scatter_desc(i - 1, bp).start()

        for bb in range(_NB):
            @pl.when(lax.rem(i, _NB) == bb)
            def _(bb=bb):
                stage(bb)

        return carry

    lax.fori_loop(0, _NCHUNK + 1, step, 0)
    for i in range(_NCHUNK - _NB, _NCHUNK):
        _scatter_desc(i, i % _NB).wait()


def kernel(tokens, embedding_table):
    pe = jnp.asarray(_PE)
    c = _build_combined(pe, embedding_table)
    out = _sc_gather(tokens.reshape(_ROWS), c)
    return out.reshape(_B, _L, _D)


# speedup vs baseline: 1.0044x; 1.0044x over previous
import functools

import numpy as np

import jax
import jax.numpy as jnp
from jax import lax
from jax.experimental import pallas as pl
from jax.experimental.pallas import tpu as pltpu
from jax.experimental.pallas import tpu_sc as plsc

_VOCAB = 42
_VPAD = 48
_D = 128
_L = 200
_B = 1024
_NC, _NS = 2, 16
_NW = _NC * _NS
_ROWS = _B * _L
_RPW = _ROWS // _NW
_K = 128
_NCHUNK = _RPW // _K
_NB = 2
_E = 4


def _pos_encoding_np(max_seq, d_model):
    even_i = np.arange(0, d_model, 2, dtype=np.float32)
    denominator = np.power(np.float32(10000.0), even_i / np.float32(d_model))
    position = np.arange(max_seq, dtype=np.float32).reshape(max_seq, 1)
    even_pe = np.sin(position / denominator, dtype=np.float32)
    odd_pe = np.cos(position / denominator, dtype=np.float32)
    stacked = np.stack([even_pe, odd_pe], axis=2)
    return stacked.reshape(max_seq, d_model).astype(np.float32)


_PE = _pos_encoding_np(_L, _D)


def _combine_body(pe_ref, tab_ref, c_ref):
    pe = pe_ref[...]
    tab = tab_ref[...]
    c_ref[...] = pe[:, None, :] + tab[None, :, :]


def _build_combined(pe, tab):
    c = pl.pallas_call(
        _combine_body,
        grid=(1,),
        in_specs=[
            pl.BlockSpec((_L, _D), lambda i: (0, 0)),
            pl.BlockSpec((_VPAD, _D), lambda i: (0, 0)),
        ],
        out_specs=pl.BlockSpec((_L, _VPAD, _D), lambda i: (0, 0, 0)),
        out_shape=jax.ShapeDtypeStruct((_L, _VPAD, _D), jnp.float32),
    )(pe, tab)
    return c.reshape(_L * _VPAD, _D)


@functools.partial(
    pl.kernel,
    out_type=jax.ShapeDtypeStruct((_ROWS, _D), jnp.float32),
    mesh=plsc.VectorSubcoreMesh(
        core_axis_name="c", subcore_axis_name="s",
        num_cores=_NC, num_subcores=_NS),
    scratch_types=[
        pltpu.VMEM((_RPW,), jnp.int32),
        pltpu.VMEM((_NB, 1, 128), jnp.int32),
        pltpu.VMEM((_NB, _K, _D), jnp.float32),
        pltpu.VMEM_SHARED((_L * _VPAD, _D), jnp.float32),
        pltpu.SemaphoreType.DMA,
        pltpu.SemaphoreType.DMA,
        pltpu.SemaphoreType.DMA,
        pltpu.SemaphoreType.DMA,
        pltpu.SemaphoreType.DMA,
    ],
)
def _sc_gather(tok_hbm, c_hbm, out_hbm, tok_v, idx_v, rows_v, c_sp,
               g0, g1, s0, s1, st):
    wid = lax.axis_index("s") * _NC + lax.axis_index("c")
    wbase = wid * _RPW
    g_sem = (g0, g1)
    s_sem = (s0, s1)

    sid = lax.axis_index("s")
    tok_cp = pltpu.make_async_copy(
        tok_hbm.at[pl.ds(wbase, _RPW)], tok_v, g_sem[0])
    tok_cp.start()

    rows_per_sub = (_L * _VPAD) // _NS
    stage_cp = pltpu.make_async_copy(
        c_hbm.at[pl.ds(sid * rows_per_sub, rows_per_sub)],
        c_sp.at[pl.ds(sid * rows_per_sub, rows_per_sub)], st)
    stage_cp.start()
    tok_cp.wait()

    def _gather_hbm_desc(i, b):
        return pltpu.make_async_copy(
            c_hbm.at[idx_v.at[b].at[0]], rows_v.at[b], g_sem[b])

    def _gather_desc(i, b):
        return pltpu.make_async_copy(
            c_sp.at[idx_v.at[b].at[0]], rows_v.at[b], g_sem[b])

    def _scatter_desc(i, b):
        return pltpu.make_async_copy(
            rows_v.at[b], out_hbm.at[pl.ds(wbase + i * _K, _K)], s_sem[b])

    def step(i, carry):
        def stage(b):
            @pl.when(jnp.logical_and(i >= _NB, i < _NCHUNK))
            def _():
                _scatter_desc(i - _NB, b).wait()

            @pl.when(i == _E)
            def _():
                stage_cp.wait()
                plsc.subcore_barrier()

            @pl.when(i < _NCHUNK)
            def _():
                for j in range(8):
                    o = i * _K + j * 16
                    pos = wbase + o + lax.iota(jnp.int32, 16)
                    l = lax.rem(pos, _L)
                    idx_v[b, 0, pl.ds(j * 16, 16)] = (
                        l * _VPAD + tok_v[pl.ds(o, 16)])

                @pl.when(i < _E)
                def _():
                    _gather_hbm_desc(i, b).start()

                @pl.when(i >= _E)
                def _():
                    _gather_desc(i, b).start()

            @pl.when(i >= 1)
            def _():
                bp = (b + _NB - 1) % _NB
                _gather_desc(i - 1, bp).wait()
                _scatter_desc(i - 1, bp).start()

        for bb in range(_NB):
            @pl.when(lax.rem(i, _NB) == bb)
            def _(bb=bb):
                stage(bb)

        return carry

    lax.fori_loop(0, _NCHUNK + 1, step, 0)
    for i in range(_NCHUNK - _NB, _NCHUNK):
        _scatter_desc(i, i % _NB).wait()


def kernel(tokens, embedding_table):
    pe = jnp.asarray(_PE)
    c = _build_combined(pe, embedding_table)
    out = _sc_gather(tokens.reshape(_ROWS), c)
    return out.reshape(_B, _L, _D)
